# baseline (speedup 1.0000x reference)
import functools

import jax
import jax.numpy as jnp
from jax import lax
from jax.experimental import pallas as pl
from jax.experimental.pallas import tpu as pltpu
from jax.experimental.pallas import tpu_sc as plsc

N = 10000
E = 320000
D = 128
NC, NS = 2, 16
C = 128
NCHUNK = 80
EP = NC * NS * NCHUNK * C
NP = 10240
RPT = NP // NS

_MESH = dict(core_axis_name="c", subcore_axis_name="s",
             num_cores=NC, num_subcores=NS)


def _sc_agg_body(sel, h_hbm, src_hbm, dst_hbm, zeros_hbm, acc_out,
                 src_v0, dst_v0, rows_v0, src_v1, dst_v1, rows_v1,
                 acc_sh, sem0, sem1):
    cid = lax.axis_index("c")
    sid = lax.axis_index("s")
    nch = 2 * NCHUNK
    ebase = sid * nch * C

    pltpu.sync_copy(zeros_hbm, acc_sh.at[pl.ds(sid * RPT, RPT)])
    plsc.subcore_barrier()

    bufs = ((src_v0, dst_v0, rows_v0, sem0), (src_v1, dst_v1, rows_v1, sem1))

    def fetch(j, buf):
        src_v, dst_v, rows_v, sem = buf
        pltpu.sync_copy(src_hbm.at[pl.ds(ebase + j * C, C)], src_v)
        pltpu.sync_copy(dst_hbm.at[pl.ds(ebase + j * C, C)], dst_v)
        pltpu.async_copy(h_hbm.at[src_v], rows_v, sem)

    def drain_scatter(buf):
        src_v, dst_v, rows_v, sem = buf
        pltpu.make_async_copy(h_hbm.at[src_v], rows_v, sem).wait()
        pltpu.sync_copy(rows_v, acc_sh.at[dst_v], add=True)

    @pl.when(cid == sel)
    def _():
        fetch(0, bufs[0])

        def body(jj, carry):
            j0 = 2 * jj
            fetch(j0 + 1, bufs[1])
            drain_scatter(bufs[0])

            @pl.when(j0 + 2 < nch)
            def _():
                fetch(j0 + 2, bufs[0])

            drain_scatter(bufs[1])
            return carry

        lax.fori_loop(0, nch // 2, body, 0)

    plsc.subcore_barrier()

    pltpu.sync_copy(acc_sh.at[pl.ds(sid * RPT, RPT)],
                    acc_out.at[cid, pl.ds(sid * RPT, RPT)])


def _make_sc_agg(sel):
    return pl.kernel(
        functools.partial(_sc_agg_body, sel),
        out_type=jax.ShapeDtypeStruct((NC, NP, D), jnp.float32),
        mesh=plsc.VectorSubcoreMesh(**_MESH),
        scratch_types=[
            pltpu.VMEM((C,), jnp.int32),
            pltpu.VMEM((C,), jnp.int32),
            pltpu.VMEM((C, D), jnp.float32),
            pltpu.VMEM((C,), jnp.int32),
            pltpu.VMEM((C,), jnp.int32),
            pltpu.VMEM((C, D), jnp.float32),
            pltpu.VMEM_SHARED((NP, D), jnp.float32),
            pltpu.SemaphoreType.DMA,
            pltpu.SemaphoreType.DMA,
        ],
    )


_sc_agg0 = _make_sc_agg(0)
_sc_agg1 = _make_sc_agg(1)


def _sc_deg_body(dst_hbm, zeros_hbm, ones_hbm, deg_out,
                 dst_v, ones_v, deg_sh):
    cid = lax.axis_index("c")
    sid = lax.axis_index("s")
    ebase = (cid * NS + sid) * NCHUNK * C

    pltpu.sync_copy(zeros_hbm, deg_sh.at[pl.ds(sid * RPT, RPT)])
    pltpu.sync_copy(ones_hbm, ones_v)
    plsc.subcore_barrier()

    def body(j, carry):
        pltpu.sync_copy(dst_hbm.at[pl.ds(ebase + j * C, C)], dst_v)
        pltpu.sync_copy(ones_v, deg_sh.at[dst_v], add=True)
        return carry

    lax.fori_loop(0, NCHUNK, body, 0)
    plsc.subcore_barrier()

    pltpu.sync_copy(deg_sh.at[pl.ds(sid * RPT, RPT)],
                    deg_out.at[cid, pl.ds(sid * RPT, RPT)])


_sc_deg = pl.kernel(
    _sc_deg_body,
    out_type=jax.ShapeDtypeStruct((NC, NP, D), jnp.float32),
    mesh=plsc.VectorSubcoreMesh(**_MESH),
    scratch_types=[
        pltpu.VMEM((C,), jnp.int32),
        pltpu.VMEM((C, D), jnp.float32),
        pltpu.VMEM_SHARED((NP, D), jnp.float32),
    ],
)


def _tc_layer_body(h_ref, acc_ref, deg_ref, wl_ref, b_ref, wr_ref, o_ref):
    deg = deg_ref[0, :, 0:1] + deg_ref[1, :, 0:1]
    mean = (acc_ref[0] + acc_ref[1]) * (1.0 / jnp.maximum(deg, 1.0))
    o = (jnp.dot(mean, wl_ref[...], preferred_element_type=jnp.float32)
         + b_ref[...]
         + jnp.dot(h_ref[...], wr_ref[...], preferred_element_type=jnp.float32))
    o_ref[...] = jnp.maximum(o, 0.0)


_TC_R = 1000


def _tc_layer(h, acc, deg, wl_t, b, wr_t):
    return pl.pallas_call(
        _tc_layer_body,
        grid=(N // _TC_R,),
        in_specs=[
            pl.BlockSpec((_TC_R, D), lambda i: (i, 0)),
            pl.BlockSpec((NC, _TC_R, D), lambda i: (0, i, 0)),
            pl.BlockSpec((NC, _TC_R, D), lambda i: (0, i, 0)),
            pl.BlockSpec((D, D), lambda i: (0, 0)),
            pl.BlockSpec((1, D), lambda i: (0, 0)),
            pl.BlockSpec((D, D), lambda i: (0, 0)),
        ],
        out_specs=pl.BlockSpec((_TC_R, D), lambda i: (i, 0)),
        out_shape=jax.ShapeDtypeStruct((N, D), jnp.float32),
    )(h, acc, deg, wl_t, b, wr_t)


def kernel(x, edge_index, W_l0, b_l0, W_r0, W_l1, b_l1, W_r1):
    src = edge_index[0].astype(jnp.int32)
    dst = edge_index[1].astype(jnp.int32)
    src = jnp.concatenate([src, jnp.zeros((EP - E,), jnp.int32)])
    dst = jnp.concatenate([dst, jnp.full((EP - E,), N, jnp.int32)])
    zeros = jnp.zeros((RPT, D), jnp.float32)
    ones = jnp.ones((C, D), jnp.float32)

    deg = _sc_deg(dst, zeros, ones)
    acc0 = _sc_agg0(x, src, dst, zeros)
    h1 = _tc_layer(x, acc0, deg, W_l0.T, b_l0.reshape(1, D), W_r0.T)
    acc1 = _sc_agg1(h1, src, dst, zeros)
    out = _tc_layer(h1, acc1, deg, W_l1.T, b_l1.reshape(1, D), W_r1.T)
    return out


# speedup vs baseline: 1.1562x; 1.1562x over previous
import jax
import jax.numpy as jnp
from jax import lax
from jax.experimental import pallas as pl
from jax.experimental.pallas import tpu as pltpu
from jax.experimental.pallas import tpu_sc as plsc

N = 10000
E = 320000
D = 128
NC, NS = 2, 16
C = 128
NCHUNK = 80
NIB = 4
NRB = 2
EP = NC * NS * NCHUNK * C
NP = 10240
RPT = NP // NS

_MESH = dict(core_axis_name="c", subcore_axis_name="s",
             num_cores=NC, num_subcores=NS)


def _sc_agg_body(h_hbm, src_hbm, dst_hbm, zeros_hbm, acc_out, *refs):
    srcs = refs[0:NIB]
    dsts = refs[NIB:2 * NIB]
    rows = refs[2 * NIB:2 * NIB + NRB]
    acc_sh = refs[2 * NIB + NRB]
    isems = refs[2 * NIB + NRB + 1:2 * NIB + NRB + 1 + NIB]
    rsems = refs[2 * NIB + NRB + 1 + NIB:]
    cid = lax.axis_index("c")
    sid = lax.axis_index("s")
    ebase = (cid * NS + sid) * NCHUNK * C

    pltpu.sync_copy(zeros_hbm, acc_sh.at[pl.ds(sid * RPT, RPT)])
    plsc.subcore_barrier()

    def fetch_idx(j, b):
        sl = pl.ds(ebase + j * C, C)
        pltpu.sync_copy(src_hbm.at[sl], srcs[b])
        pltpu.sync_copy(dst_hbm.at[sl], dsts[b])

    def start_gather(j, b, rb):
        pltpu.async_copy(h_hbm.at[srcs[b]], rows[rb], rsems[rb])

    def drain_scatter(b, rb):
        pltpu.make_async_copy(h_hbm.at[srcs[b]], rows[rb], rsems[rb]).wait()
        pltpu.sync_copy(rows[rb], acc_sh.at[dsts[b]], add=True)

    for r in range(NIB - 1):
        fetch_idx(r, r)
    start_gather(0, 0, 0)

    def body(jj, carry):
        for r in range(NIB):
            j = jj * NIB + r

            @pl.when(j + NIB - 1 < NCHUNK)
            def _():
                fetch_idx(j + NIB - 1, (r + NIB - 1) % NIB)

            @pl.when(j + 1 < NCHUNK)
            def _():
                start_gather(j + 1, (r + 1) % NIB, (r + 1) % NRB)

            drain_scatter(r, r % NRB)
        return carry

    lax.fori_loop(0, NCHUNK // NIB, body, 0)
    plsc.subcore_barrier()

    pltpu.sync_copy(acc_sh.at[pl.ds(sid * RPT, RPT)],
                    acc_out.at[cid, pl.ds(sid * RPT, RPT)])


_sc_agg = pl.kernel(
    _sc_agg_body,
    out_type=jax.ShapeDtypeStruct((NC, NP, D), jnp.float32),
    mesh=plsc.VectorSubcoreMesh(**_MESH),
    scratch_types=(
        [pltpu.VMEM((C,), jnp.int32)] * NIB
        + [pltpu.VMEM((C,), jnp.int32)] * NIB
        + [pltpu.VMEM((C, D), jnp.float32)] * NRB
        + [pltpu.VMEM_SHARED((NP, D), jnp.float32)]
        + [pltpu.SemaphoreType.DMA] * (NIB + NRB)
    ),
)


def _sc_deg_body(dst_hbm, zeros_hbm, ones_hbm, deg_out,
                 dst_v, ones_v, deg_sh):
    cid = lax.axis_index("c")
    sid = lax.axis_index("s")
    ebase = (cid * NS + sid) * NCHUNK * C

    pltpu.sync_copy(zeros_hbm, deg_sh.at[pl.ds(sid * RPT, RPT)])
    pltpu.sync_copy(ones_hbm, ones_v)
    plsc.subcore_barrier()

    def body(j, carry):
        pltpu.sync_copy(dst_hbm.at[pl.ds(ebase + j * C, C)], dst_v)
        pltpu.sync_copy(ones_v, deg_sh.at[dst_v], add=True)
        return carry

    lax.fori_loop(0, NCHUNK, body, 0)
    plsc.subcore_barrier()

    pltpu.sync_copy(deg_sh.at[pl.ds(sid * RPT, RPT)],
                    deg_out.at[cid, pl.ds(sid * RPT, RPT)])


_sc_deg = pl.kernel(
    _sc_deg_body,
    out_type=jax.ShapeDtypeStruct((NC, NP, D), jnp.float32),
    mesh=plsc.VectorSubcoreMesh(**_MESH),
    scratch_types=[
        pltpu.VMEM((C,), jnp.int32),
        pltpu.VMEM((C, D), jnp.float32),
        pltpu.VMEM_SHARED((NP, D), jnp.float32),
    ],
)


def _tc_layer_body(h_ref, acc_ref, deg_ref, wl_ref, b_ref, wr_ref, o_ref):
    deg = deg_ref[0, :, 0:1] + deg_ref[1, :, 0:1]
    mean = (acc_ref[0] + acc_ref[1]) * (1.0 / jnp.maximum(deg, 1.0))
    o = (jnp.dot(mean, wl_ref[...], preferred_element_type=jnp.float32)
         + b_ref[...]
         + jnp.dot(h_ref[...], wr_ref[...], preferred_element_type=jnp.float32))
    o_ref[...] = jnp.maximum(o, 0.0)


_TC_R = 1000


def _tc_layer(h, acc, deg, wl_t, b, wr_t):
    return pl.pallas_call(
        _tc_layer_body,
        grid=(N // _TC_R,),
        in_specs=[
            pl.BlockSpec((_TC_R, D), lambda i: (i, 0)),
            pl.BlockSpec((NC, _TC_R, D), lambda i: (0, i, 0)),
            pl.BlockSpec((NC, _TC_R, D), lambda i: (0, i, 0)),
            pl.BlockSpec((D, D), lambda i: (0, 0)),
            pl.BlockSpec((1, D), lambda i: (0, 0)),
            pl.BlockSpec((D, D), lambda i: (0, 0)),
        ],
        out_specs=pl.BlockSpec((_TC_R, D), lambda i: (i, 0)),
        out_shape=jax.ShapeDtypeStruct((N, D), jnp.float32),
    )(h, acc, deg, wl_t, b, wr_t)


def kernel(x, edge_index, W_l0, b_l0, W_r0, W_l1, b_l1, W_r1):
    src = edge_index[0].astype(jnp.int32)
    dst = edge_index[1].astype(jnp.int32)
    src = jnp.concatenate([src, jnp.zeros((EP - E,), jnp.int32)])
    dst = jnp.concatenate([dst, jnp.full((EP - E,), N, jnp.int32)])
    zeros = jnp.zeros((RPT, D), jnp.float32)
    ones = jnp.ones((C, D), jnp.float32)

    deg = _sc_deg(dst, zeros, ones)
    acc0 = _sc_agg(x, src, dst, zeros)
    h1 = _tc_layer(x, acc0, deg, W_l0.T, b_l0.reshape(1, D), W_r0.T)
    acc1 = _sc_agg(h1, src, dst, zeros)
    out = _tc_layer(h1, acc1, deg, W_l1.T, b_l1.reshape(1, D), W_r1.T)
    return out


# speedup vs baseline: 1.1603x; 1.0035x over previous
import jax
import jax.numpy as jnp
from jax import lax
from jax.experimental import pallas as pl
from jax.experimental.pallas import tpu as pltpu
from jax.experimental.pallas import tpu_sc as plsc

N = 10000
E = 320000
D = 128
NC, NS = 2, 16
C = 128
NCHUNK = 80
EP = NC * NS * NCHUNK * C
NP = 10240
RPT = NP // NS

_MESH = dict(core_axis_name="c", subcore_axis_name="s",
             num_cores=NC, num_subcores=NS)


def _sc_agg_body(h_hbm, src_hbm, dst_hbm, zeros_hbm, acc_out,
                 src_v0, dst_v0, rows_v0, src_v1, dst_v1, rows_v1,
                 acc_sh, gsem0, gsem1, ssem0, ssem1):
    cid = lax.axis_index("c")
    sid = lax.axis_index("s")
    ebase = (cid * NS + sid) * NCHUNK * C

    pltpu.sync_copy(zeros_hbm, acc_sh.at[pl.ds(sid * RPT, RPT)])
    plsc.subcore_barrier()

    bufs = ((src_v0, dst_v0, rows_v0, gsem0, ssem0),
            (src_v1, dst_v1, rows_v1, gsem1, ssem1))

    def fetch(j, buf):
        src_v, dst_v, rows_v, gsem, _ = buf
        pltpu.sync_copy(src_hbm.at[pl.ds(ebase + j * C, C)], src_v)
        pltpu.sync_copy(dst_hbm.at[pl.ds(ebase + j * C, C)], dst_v)
        pltpu.async_copy(h_hbm.at[src_v], rows_v, gsem)

    def start_scatter(buf):
        src_v, dst_v, rows_v, gsem, ssem = buf
        pltpu.make_async_copy(h_hbm.at[src_v], rows_v, gsem).wait()
        pltpu.async_copy(rows_v, acc_sh.at[dst_v], ssem, add=True)

    def wait_scatter(buf):
        src_v, dst_v, rows_v, _, ssem = buf
        pltpu.make_async_copy(rows_v, acc_sh.at[dst_v], ssem).wait()

    fetch(0, bufs[0])
    fetch(1, bufs[1])

    def body(jj, carry):
        j0 = 2 * jj
        start_scatter(bufs[0])

        @pl.when(j0 + 2 < NCHUNK)
        def _():
            wait_scatter(bufs[0])
            fetch(j0 + 2, bufs[0])

        start_scatter(bufs[1])

        @pl.when(j0 + 3 < NCHUNK)
        def _():
            wait_scatter(bufs[1])
            fetch(j0 + 3, bufs[1])

        return carry

    lax.fori_loop(0, NCHUNK // 2, body, 0)
    wait_scatter(bufs[0])
    wait_scatter(bufs[1])
    plsc.subcore_barrier()

    pltpu.sync_copy(acc_sh.at[pl.ds(sid * RPT, RPT)],
                    acc_out.at[cid, pl.ds(sid * RPT, RPT)])


_sc_agg = pl.kernel(
    _sc_agg_body,
    out_type=jax.ShapeDtypeStruct((NC, NP, D), jnp.float32),
    mesh=plsc.VectorSubcoreMesh(**_MESH),
    scratch_types=[
        pltpu.VMEM((C,), jnp.int32),
        pltpu.VMEM((C,), jnp.int32),
        pltpu.VMEM((C, D), jnp.float32),
        pltpu.VMEM((C,), jnp.int32),
        pltpu.VMEM((C,), jnp.int32),
        pltpu.VMEM((C, D), jnp.float32),
        pltpu.VMEM_SHARED((NP, D), jnp.float32),
        pltpu.SemaphoreType.DMA,
        pltpu.SemaphoreType.DMA,
        pltpu.SemaphoreType.DMA,
        pltpu.SemaphoreType.DMA,
    ],
)


def _sc_deg_body(dst_hbm, zeros_hbm, ones_hbm, deg_out,
                 dst_v, ones_v, deg_sh):
    cid = lax.axis_index("c")
    sid = lax.axis_index("s")
    ebase = (cid * NS + sid) * NCHUNK * C

    pltpu.sync_copy(zeros_hbm, deg_sh.at[pl.ds(sid * RPT, RPT)])
    pltpu.sync_copy(ones_hbm, ones_v)
    plsc.subcore_barrier()

    def body(j, carry):
        pltpu.sync_copy(dst_hbm.at[pl.ds(ebase + j * C, C)], dst_v)
        pltpu.sync_copy(ones_v, deg_sh.at[dst_v], add=True)
        return carry

    lax.fori_loop(0, NCHUNK, body, 0)
    plsc.subcore_barrier()

    pltpu.sync_copy(deg_sh.at[pl.ds(sid * RPT, RPT)],
                    deg_out.at[cid, pl.ds(sid * RPT, RPT)])


_sc_deg = pl.kernel(
    _sc_deg_body,
    out_type=jax.ShapeDtypeStruct((NC, NP, D), jnp.float32),
    mesh=plsc.VectorSubcoreMesh(**_MESH),
    scratch_types=[
        pltpu.VMEM((C,), jnp.int32),
        pltpu.VMEM((C, D), jnp.float32),
        pltpu.VMEM_SHARED((NP, D), jnp.float32),
    ],
)


def _tc_layer_body(h_ref, acc_ref, deg_ref, wl_ref, b_ref, wr_ref, o_ref):
    deg = deg_ref[0, :, 0:1] + deg_ref[1, :, 0:1]
    mean = (acc_ref[0] + acc_ref[1]) * (1.0 / jnp.maximum(deg, 1.0))
    o = (jnp.dot(mean, wl_ref[...], preferred_element_type=jnp.float32)
         + b_ref[...]
         + jnp.dot(h_ref[...], wr_ref[...], preferred_element_type=jnp.float32))
    o_ref[...] = jnp.maximum(o, 0.0)


_TC_R = 1000


def _tc_layer(h, acc, deg, wl_t, b, wr_t):
    return pl.pallas_call(
        _tc_layer_body,
        grid=(N // _TC_R,),
        in_specs=[
            pl.BlockSpec((_TC_R, D), lambda i: (i, 0)),
            pl.BlockSpec((NC, _TC_R, D), lambda i: (0, i, 0)),
            pl.BlockSpec((NC, _TC_R, D), lambda i: (0, i, 0)),
            pl.BlockSpec((D, D), lambda i: (0, 0)),
            pl.BlockSpec((1, D), lambda i: (0, 0)),
            pl.BlockSpec((D, D), lambda i: (0, 0)),
        ],
        out_specs=pl.BlockSpec((_TC_R, D), lambda i: (i, 0)),
        out_shape=jax.ShapeDtypeStruct((N, D), jnp.float32),
    )(h, acc, deg, wl_t, b, wr_t)


def kernel(x, edge_index, W_l0, b_l0, W_r0, W_l1, b_l1, W_r1):
    src = edge_index[0].astype(jnp.int32)
    dst = edge_index[1].astype(jnp.int32)
    src = jnp.concatenate([src, jnp.zeros((EP - E,), jnp.int32)])
    dst = jnp.concatenate([dst, jnp.full((EP - E,), N, jnp.int32)])
    zeros = jnp.zeros((RPT, D), jnp.float32)
    ones = jnp.ones((C, D), jnp.float32)

    deg = _sc_deg(dst, zeros, ones)
    acc0 = _sc_agg(x, src, dst, zeros)
    h1 = _tc_layer(x, acc0, deg, W_l0.T, b_l0.reshape(1, D), W_r0.T)
    acc1 = _sc_agg(h1, src, dst, zeros)
    out = _tc_layer(h1, acc1, deg, W_l1.T, b_l1.reshape(1, D), W_r1.T)
    return out


# speedup vs baseline: 1.2269x; 1.0574x over previous
import jax
import jax.numpy as jnp
from jax import lax
from jax.experimental import pallas as pl
from jax.experimental.pallas import tpu as pltpu
from jax.experimental.pallas import tpu_sc as plsc

N = 10000
E = 320000
D = 128
NC, NS = 2, 16
C = 128
NCHUNK = 80
EP = NC * NS * NCHUNK * C
NP = 10240
RPT = NP // NS
ZR = 80

_MESH = dict(core_axis_name="c", subcore_axis_name="s",
             num_cores=NC, num_subcores=NS)


def _sc_agg_body(h_hbm, src_hbm, dst_hbm, zeros_hbm, acc_out,
                 src_v0, dst_v0, rows_v0, src_v1, dst_v1, rows_v1,
                 zrow_v, acc_sh, gsem0, gsem1, ssem0, ssem1):
    cid = lax.axis_index("c")
    sid = lax.axis_index("s")
    ebase = (cid * NS + sid) * NCHUNK * C

    pltpu.sync_copy(zeros_hbm, zrow_v)
    for k in range(RPT // ZR):
        pltpu.sync_copy(zrow_v, acc_sh.at[pl.ds(sid * RPT + k * ZR, ZR)])
    plsc.subcore_barrier()

    bufs = ((src_v0, dst_v0, rows_v0, gsem0, ssem0),
            (src_v1, dst_v1, rows_v1, gsem1, ssem1))

    def fetch(j, buf):
        src_v, dst_v, rows_v, gsem, _ = buf
        pltpu.sync_copy(src_hbm.at[pl.ds(ebase + j * C, C)], src_v)
        pltpu.sync_copy(dst_hbm.at[pl.ds(ebase + j * C, C)], dst_v)
        pltpu.async_copy(h_hbm.at[src_v], rows_v, gsem)

    def start_scatter(buf):
        src_v, dst_v, rows_v, gsem, ssem = buf
        pltpu.make_async_copy(h_hbm.at[src_v], rows_v, gsem).wait()
        pltpu.async_copy(rows_v, acc_sh.at[dst_v], ssem, add=True)

    def wait_scatter(buf):
        src_v, dst_v, rows_v, _, ssem = buf
        pltpu.make_async_copy(rows_v, acc_sh.at[dst_v], ssem).wait()

    fetch(0, bufs[0])
    fetch(1, bufs[1])

    def body(jj, carry):
        j0 = 2 * jj
        start_scatter(bufs[0])

        @pl.when(j0 + 2 < NCHUNK)
        def _():
            wait_scatter(bufs[0])
            fetch(j0 + 2, bufs[0])

        start_scatter(bufs[1])

        @pl.when(j0 + 3 < NCHUNK)
        def _():
            wait_scatter(bufs[1])
            fetch(j0 + 3, bufs[1])

        return carry

    lax.fori_loop(0, NCHUNK // 2, body, 0)
    wait_scatter(bufs[0])
    wait_scatter(bufs[1])
    plsc.subcore_barrier()

    pltpu.sync_copy(acc_sh.at[pl.ds(sid * RPT, RPT)],
                    acc_out.at[cid, pl.ds(sid * RPT, RPT)])


_sc_agg = pl.kernel(
    _sc_agg_body,
    out_type=jax.ShapeDtypeStruct((NC, NP, D), jnp.float32),
    mesh=plsc.VectorSubcoreMesh(**_MESH),
    scratch_types=[
        pltpu.VMEM((C,), jnp.int32),
        pltpu.VMEM((C,), jnp.int32),
        pltpu.VMEM((C, D), jnp.float32),
        pltpu.VMEM((C,), jnp.int32),
        pltpu.VMEM((C,), jnp.int32),
        pltpu.VMEM((C, D), jnp.float32),
        pltpu.VMEM((ZR, D), jnp.float32),
        pltpu.VMEM_SHARED((NP, D), jnp.float32),
        pltpu.SemaphoreType.DMA,
        pltpu.SemaphoreType.DMA,
        pltpu.SemaphoreType.DMA,
        pltpu.SemaphoreType.DMA,
    ],
)


def _sc_deg_body(dst_hbm, zeros_hbm, ones_hbm, deg_out,
                 dst_v, ones_v, deg_sh):
    cid = lax.axis_index("c")
    sid = lax.axis_index("s")
    ebase = (cid * NS + sid) * NCHUNK * C

    pltpu.sync_copy(zeros_hbm, ones_v.at[pl.ds(0, ZR)])
    for k in range(RPT // ZR):
        pltpu.sync_copy(ones_v.at[pl.ds(0, ZR)],
                        deg_sh.at[pl.ds(sid * RPT + k * ZR, ZR)])
    pltpu.sync_copy(ones_hbm, ones_v)
    plsc.subcore_barrier()

    def body(j, carry):
        pltpu.sync_copy(dst_hbm.at[pl.ds(ebase + j * C, C)], dst_v)
        pltpu.sync_copy(ones_v, deg_sh.at[dst_v], add=True)
        return carry

    lax.fori_loop(0, NCHUNK, body, 0)
    plsc.subcore_barrier()

    pltpu.sync_copy(deg_sh.at[pl.ds(sid * RPT, RPT)],
                    deg_out.at[cid, pl.ds(sid * RPT, RPT)])


_sc_deg = pl.kernel(
    _sc_deg_body,
    out_type=jax.ShapeDtypeStruct((NC, NP, D), jnp.float32),
    mesh=plsc.VectorSubcoreMesh(**_MESH),
    scratch_types=[
        pltpu.VMEM((C,), jnp.int32),
        pltpu.VMEM((C, D), jnp.float32),
        pltpu.VMEM_SHARED((NP, D), jnp.float32),
    ],
)


def _tc_layer_body(h_ref, acc_ref, deg_ref, wl_ref, b_ref, wr_ref, o_ref):
    deg = deg_ref[0, :, 0:1] + deg_ref[1, :, 0:1]
    mean = (acc_ref[0] + acc_ref[1]) * (1.0 / jnp.maximum(deg, 1.0))
    o = (jnp.dot(mean, wl_ref[...], preferred_element_type=jnp.float32)
         + b_ref[...]
         + jnp.dot(h_ref[...], wr_ref[...], preferred_element_type=jnp.float32))
    o_ref[...] = jnp.maximum(o, 0.0)


_TC_R = 1000


def _tc_layer(h, acc, deg, wl_t, b, wr_t):
    return pl.pallas_call(
        _tc_layer_body,
        grid=(N // _TC_R,),
        in_specs=[
            pl.BlockSpec((_TC_R, D), lambda i: (i, 0)),
            pl.BlockSpec((NC, _TC_R, D), lambda i: (0, i, 0)),
            pl.BlockSpec((NC, _TC_R, D), lambda i: (0, i, 0)),
            pl.BlockSpec((D, D), lambda i: (0, 0)),
            pl.BlockSpec((1, D), lambda i: (0, 0)),
            pl.BlockSpec((D, D), lambda i: (0, 0)),
        ],
        out_specs=pl.BlockSpec((_TC_R, D), lambda i: (i, 0)),
        out_shape=jax.ShapeDtypeStruct((N, D), jnp.float32),
    )(h, acc, deg, wl_t, b, wr_t)


def kernel(x, edge_index, W_l0, b_l0, W_r0, W_l1, b_l1, W_r1):
    src = edge_index[0].astype(jnp.int32)
    dst = edge_index[1].astype(jnp.int32)
    src = jnp.concatenate([src, jnp.zeros((EP - E,), jnp.int32)])
    dst = jnp.concatenate([dst, jnp.full((EP - E,), N, jnp.int32)])
    zeros = jnp.zeros((ZR, D), jnp.float32)
    ones = jnp.ones((C, D), jnp.float32)

    deg = _sc_deg(dst, zeros, ones)
    acc0 = _sc_agg(x, src, dst, zeros)
    h1 = _tc_layer(x, acc0, deg, W_l0.T, b_l0.reshape(1, D), W_r0.T)
    acc1 = _sc_agg(h1, src, dst, zeros)
    out = _tc_layer(h1, acc1, deg, W_l1.T, b_l1.reshape(1, D), W_r1.T)
    return out


# speedup vs baseline: 1.2676x; 1.0332x over previous
import jax
import jax.numpy as jnp
from jax import lax
from jax.experimental import pallas as pl
from jax.experimental.pallas import tpu as pltpu
from jax.experimental.pallas import tpu_sc as plsc

N = 10000
E = 320000
D = 128
NC, NS = 2, 16
C = 128
NCHUNK = 80
EP = NC * NS * NCHUNK * C
NP = 10240
RPT = NP // NS
ZR = 80

_MESH = dict(core_axis_name="c", subcore_axis_name="s",
             num_cores=NC, num_subcores=NS)


def _sc_agg_body(h_hbm, src_hbm, dst_hbm, zeros_hbm, acc_out,
                 src_v0, dst_v0, rows_v0, src_v1, dst_v1, rows_v1,
                 zrow_v, acc_sh, gsem0, gsem1, ssem0, ssem1):
    cid = lax.axis_index("c")
    sid = lax.axis_index("s")
    ebase = (cid * NS + sid) * NCHUNK * C

    pltpu.sync_copy(zeros_hbm, zrow_v)
    for k in range(RPT // ZR):
        pltpu.sync_copy(zrow_v, acc_sh.at[pl.ds(sid * RPT + k * ZR, ZR)])
    plsc.subcore_barrier()

    bufs = ((src_v0, dst_v0, rows_v0, gsem0, ssem0),
            (src_v1, dst_v1, rows_v1, gsem1, ssem1))

    def fetch(j, buf):
        src_v, dst_v, rows_v, gsem, _ = buf
        pltpu.sync_copy(src_hbm.at[pl.ds(ebase + j * C, C)], src_v)
        pltpu.sync_copy(dst_hbm.at[pl.ds(ebase + j * C, C)], dst_v)
        pltpu.async_copy(h_hbm.at[src_v], rows_v, gsem)

    def start_scatter(buf):
        src_v, dst_v, rows_v, gsem, ssem = buf
        pltpu.make_async_copy(h_hbm.at[src_v], rows_v, gsem).wait()
        pltpu.async_copy(rows_v, acc_sh.at[dst_v], ssem, add=True)

    def wait_scatter(buf):
        src_v, dst_v, rows_v, _, ssem = buf
        pltpu.make_async_copy(rows_v, acc_sh.at[dst_v], ssem).wait()

    fetch(0, bufs[0])
    fetch(1, bufs[1])

    def body(jj, carry):
        j0 = 2 * jj
        start_scatter(bufs[0])

        @pl.when(j0 + 2 < NCHUNK)
        def _():
            wait_scatter(bufs[0])
            fetch(j0 + 2, bufs[0])

        start_scatter(bufs[1])

        @pl.when(j0 + 3 < NCHUNK)
        def _():
            wait_scatter(bufs[1])
            fetch(j0 + 3, bufs[1])

        return carry

    lax.fori_loop(0, NCHUNK // 2, body, 0)
    wait_scatter(bufs[0])
    wait_scatter(bufs[1])
    plsc.subcore_barrier()

    pltpu.sync_copy(acc_sh.at[pl.ds(sid * RPT, RPT)],
                    acc_out.at[cid, pl.ds(sid * RPT, RPT)])


_sc_agg = pl.kernel(
    _sc_agg_body,
    out_type=jax.ShapeDtypeStruct((NC, NP, D), jnp.float32),
    mesh=plsc.VectorSubcoreMesh(**_MESH),
    scratch_types=[
        pltpu.VMEM((C,), jnp.int32),
        pltpu.VMEM((C,), jnp.int32),
        pltpu.VMEM((C, D), jnp.float32),
        pltpu.VMEM((C,), jnp.int32),
        pltpu.VMEM((C,), jnp.int32),
        pltpu.VMEM((C, D), jnp.float32),
        pltpu.VMEM((ZR, D), jnp.float32),
        pltpu.VMEM_SHARED((NP, D), jnp.float32),
        pltpu.SemaphoreType.DMA,
        pltpu.SemaphoreType.DMA,
        pltpu.SemaphoreType.DMA,
        pltpu.SemaphoreType.DMA,
    ],
)


def _sc_deg_body(dst_hbm, zeros_hbm, ones_hbm, deg_out,
                 dst_v0, dst_v1, ones_v, deg_sh, ssem0, ssem1):
    cid = lax.axis_index("c")
    sid = lax.axis_index("s")
    ebase = (cid * NS + sid) * NCHUNK * C

    pltpu.sync_copy(zeros_hbm, ones_v.at[pl.ds(0, ZR)])
    for k in range(RPT // ZR):
        pltpu.sync_copy(ones_v.at[pl.ds(0, ZR)],
                        deg_sh.at[pl.ds(sid * RPT + k * ZR, ZR)])
    pltpu.sync_copy(ones_hbm, ones_v)
    plsc.subcore_barrier()

    bufs = ((dst_v0, ssem0), (dst_v1, ssem1))

    def fetch_scatter(j, buf):
        dst_v, ssem = buf
        pltpu.sync_copy(dst_hbm.at[pl.ds(ebase + j * C, C)], dst_v)
        pltpu.async_copy(ones_v, deg_sh.at[dst_v], ssem, add=True)

    def wait_scatter(buf):
        dst_v, ssem = buf
        pltpu.make_async_copy(ones_v, deg_sh.at[dst_v], ssem).wait()

    fetch_scatter(0, bufs[0])

    def body(jj, carry):
        j0 = 2 * jj
        fetch_scatter(j0 + 1, bufs[1])
        wait_scatter(bufs[0])

        @pl.when(j0 + 2 < NCHUNK)
        def _():
            fetch_scatter(j0 + 2, bufs[0])

        wait_scatter(bufs[1])
        return carry

    lax.fori_loop(0, NCHUNK // 2, body, 0)
    plsc.subcore_barrier()

    pltpu.sync_copy(deg_sh.at[pl.ds(sid * RPT, RPT)],
                    deg_out.at[cid, pl.ds(sid * RPT, RPT)])


_sc_deg = pl.kernel(
    _sc_deg_body,
    out_type=jax.ShapeDtypeStruct((NC, NP, D), jnp.float32),
    mesh=plsc.VectorSubcoreMesh(**_MESH),
    scratch_types=[
        pltpu.VMEM((C,), jnp.int32),
        pltpu.VMEM((C,), jnp.int32),
        pltpu.VMEM((C, D), jnp.float32),
        pltpu.VMEM_SHARED((NP, D), jnp.float32),
        pltpu.SemaphoreType.DMA,
        pltpu.SemaphoreType.DMA,
    ],
)


def _tc_layer_body(h_ref, acc_ref, deg_ref, wl_ref, b_ref, wr_ref, o_ref):
    deg = deg_ref[0, :, 0:1] + deg_ref[1, :, 0:1]
    mean = (acc_ref[0] + acc_ref[1]) * (1.0 / jnp.maximum(deg, 1.0))
    o = (jnp.dot(mean, wl_ref[...], preferred_element_type=jnp.float32)
         + b_ref[...]
         + jnp.dot(h_ref[...], wr_ref[...], preferred_element_type=jnp.float32))
    o_ref[...] = jnp.maximum(o, 0.0)


_TC_R = 1000


def _tc_layer(h, acc, deg, wl_t, b, wr_t):
    return pl.pallas_call(
        _tc_layer_body,
        grid=(N // _TC_R,),
        in_specs=[
            pl.BlockSpec((_TC_R, D), lambda i: (i, 0)),
            pl.BlockSpec((NC, _TC_R, D), lambda i: (0, i, 0)),
            pl.BlockSpec((NC, _TC_R, D), lambda i: (0, i, 0)),
            pl.BlockSpec((D, D), lambda i: (0, 0)),
            pl.BlockSpec((1, D), lambda i: (0, 0)),
            pl.BlockSpec((D, D), lambda i: (0, 0)),
        ],
        out_specs=pl.BlockSpec((_TC_R, D), lambda i: (i, 0)),
        out_shape=jax.ShapeDtypeStruct((N, D), jnp.float32),
    )(h, acc, deg, wl_t, b, wr_t)


def kernel(x, edge_index, W_l0, b_l0, W_r0, W_l1, b_l1, W_r1):
    src = edge_index[0].astype(jnp.int32)
    dst = edge_index[1].astype(jnp.int32)
    src = jnp.concatenate([src, jnp.zeros((EP - E,), jnp.int32)])
    dst = jnp.concatenate([dst, jnp.full((EP - E,), N, jnp.int32)])
    zeros = jnp.zeros((ZR, D), jnp.float32)
    ones = jnp.ones((C, D), jnp.float32)

    deg = _sc_deg(dst, zeros, ones)
    acc0 = _sc_agg(x, src, dst, zeros)
    h1 = _tc_layer(x, acc0, deg, W_l0.T, b_l0.reshape(1, D), W_r0.T)
    acc1 = _sc_agg(h1, src, dst, zeros)
    out = _tc_layer(h1, acc1, deg, W_l1.T, b_l1.reshape(1, D), W_r1.T)
    return out


# speedup vs baseline: 1.2729x; 1.0042x over previous
import jax
import jax.numpy as jnp
from jax import lax
from jax.experimental import pallas as pl
from jax.experimental.pallas import tpu as pltpu
from jax.experimental.pallas import tpu_sc as plsc

N = 10000
E = 320000
D = 128
NC, NS = 2, 16
C = 128
NCHUNK = 80
EP = NC * NS * NCHUNK * C
NP = 10240
RPT = NP // NS
ZR = 80

_MESH = dict(core_axis_name="c", subcore_axis_name="s",
             num_cores=NC, num_subcores=NS)


def _sc_agg_body(h_hbm, src_hbm, dst_hbm, zeros_hbm, acc_out,
                 src_v0, dst_v0, rows_v0, src_v1, dst_v1, rows_v1,
                 zrow_v, acc_sh, gsem0, gsem1, ssem0, ssem1):
    cid = lax.axis_index("c")
    sid = lax.axis_index("s")
    ebase = (cid * NS + sid) * NCHUNK * C

    pltpu.sync_copy(zeros_hbm, zrow_v)
    for k in range(RPT // ZR):
        pltpu.sync_copy(zrow_v, acc_sh.at[pl.ds(sid * RPT + k * ZR, ZR)])
    plsc.subcore_barrier()

    bufs = ((src_v0, dst_v0, rows_v0, gsem0, ssem0),
            (src_v1, dst_v1, rows_v1, gsem1, ssem1))

    def fetch(j, buf):
        src_v, dst_v, rows_v, gsem, _ = buf
        pltpu.sync_copy(src_hbm.at[pl.ds(ebase + j * C, C)], src_v)
        pltpu.sync_copy(dst_hbm.at[pl.ds(ebase + j * C, C)], dst_v)
        pltpu.async_copy(h_hbm.at[src_v], rows_v, gsem)

    def start_scatter(buf):
        src_v, dst_v, rows_v, gsem, ssem = buf
        pltpu.make_async_copy(h_hbm.at[src_v], rows_v, gsem).wait()
        pltpu.async_copy(rows_v, acc_sh.at[dst_v], ssem, add=True)

    def wait_scatter(buf):
        src_v, dst_v, rows_v, _, ssem = buf
        pltpu.make_async_copy(rows_v, acc_sh.at[dst_v], ssem).wait()

    fetch(0, bufs[0])
    fetch(1, bufs[1])

    def body(jj, carry):
        j0 = 2 * jj
        start_scatter(bufs[0])

        @pl.when(j0 + 2 < NCHUNK)
        def _():
            wait_scatter(bufs[0])
            fetch(j0 + 2, bufs[0])

        start_scatter(bufs[1])

        @pl.when(j0 + 3 < NCHUNK)
        def _():
            wait_scatter(bufs[1])
            fetch(j0 + 3, bufs[1])

        return carry

    lax.fori_loop(0, NCHUNK // 2, body, 0)
    wait_scatter(bufs[0])
    wait_scatter(bufs[1])
    plsc.subcore_barrier()

    pltpu.sync_copy(acc_sh.at[pl.ds(sid * RPT, RPT)],
                    acc_out.at[cid, pl.ds(sid * RPT, RPT)])


_sc_agg = pl.kernel(
    _sc_agg_body,
    out_type=jax.ShapeDtypeStruct((NC, NP, D), jnp.float32),
    mesh=plsc.VectorSubcoreMesh(**_MESH),
    scratch_types=[
        pltpu.VMEM((C,), jnp.int32),
        pltpu.VMEM((C,), jnp.int32),
        pltpu.VMEM((C, D), jnp.float32),
        pltpu.VMEM((C,), jnp.int32),
        pltpu.VMEM((C,), jnp.int32),
        pltpu.VMEM((C, D), jnp.float32),
        pltpu.VMEM((ZR, D), jnp.float32),
        pltpu.VMEM_SHARED((NP, D), jnp.float32),
        pltpu.SemaphoreType.DMA,
        pltpu.SemaphoreType.DMA,
        pltpu.SemaphoreType.DMA,
        pltpu.SemaphoreType.DMA,
    ],
)


def _sc_deg_body(dst_hbm, zeros_hbm, ones_hbm, deg_out,
                 dst_v0, dst_v1, ones_v, deg_sh, ssem0, ssem1):
    cid = lax.axis_index("c")
    sid = lax.axis_index("s")
    ebase = (cid * NS + sid) * NCHUNK * C

    pltpu.sync_copy(zeros_hbm, ones_v.at[pl.ds(0, ZR)])
    for k in range(RPT // ZR):
        pltpu.sync_copy(ones_v.at[pl.ds(0, ZR)],
                        deg_sh.at[pl.ds(sid * RPT + k * ZR, ZR)])
    pltpu.sync_copy(ones_hbm, ones_v)
    plsc.subcore_barrier()

    bufs = ((dst_v0, ssem0), (dst_v1, ssem1))

    def fetch_scatter(j, buf):
        dst_v, ssem = buf
        pltpu.sync_copy(dst_hbm.at[pl.ds(ebase + j * C, C)], dst_v)
        pltpu.async_copy(ones_v, deg_sh.at[dst_v], ssem, add=True)

    def wait_scatter(buf):
        dst_v, ssem = buf
        pltpu.make_async_copy(ones_v, deg_sh.at[dst_v], ssem).wait()

    fetch_scatter(0, bufs[0])

    def body(jj, carry):
        j0 = 2 * jj
        fetch_scatter(j0 + 1, bufs[1])
        wait_scatter(bufs[0])

        @pl.when(j0 + 2 < NCHUNK)
        def _():
            fetch_scatter(j0 + 2, bufs[0])

        wait_scatter(bufs[1])
        return carry

    lax.fori_loop(0, NCHUNK // 2, body, 0)
    plsc.subcore_barrier()

    pltpu.sync_copy(deg_sh.at[pl.ds(sid * RPT, RPT)],
                    deg_out.at[cid, pl.ds(sid * RPT, RPT)])


_sc_deg = pl.kernel(
    _sc_deg_body,
    out_type=jax.ShapeDtypeStruct((NC, NP, D), jnp.float32),
    mesh=plsc.VectorSubcoreMesh(**_MESH),
    scratch_types=[
        pltpu.VMEM((C,), jnp.int32),
        pltpu.VMEM((C,), jnp.int32),
        pltpu.VMEM((C, D), jnp.float32),
        pltpu.VMEM_SHARED((NP, D), jnp.float32),
        pltpu.SemaphoreType.DMA,
        pltpu.SemaphoreType.DMA,
    ],
)


def _tc_layer_body(h_ref, acc_ref, deg_ref, wl_ref, b_ref, wr_ref, o_ref):
    deg = deg_ref[0, :, 0:1] + deg_ref[1, :, 0:1]
    mean = (acc_ref[0] + acc_ref[1]) * (1.0 / jnp.maximum(deg, 1.0))
    o = (jnp.dot(mean, wl_ref[...], preferred_element_type=jnp.float32)
         + b_ref[...]
         + jnp.dot(h_ref[...], wr_ref[...], preferred_element_type=jnp.float32))
    o_ref[...] = jnp.maximum(o, 0.0)


_TC_R = 2000


def _tc_layer(h, acc, deg, wl_t, b, wr_t):
    return pl.pallas_call(
        _tc_layer_body,
        grid=(N // _TC_R,),
        in_specs=[
            pl.BlockSpec((_TC_R, D), lambda i: (i, 0)),
            pl.BlockSpec((NC, _TC_R, D), lambda i: (0, i, 0)),
            pl.BlockSpec((NC, _TC_R, D), lambda i: (0, i, 0)),
            pl.BlockSpec((D, D), lambda i: (0, 0)),
            pl.BlockSpec((1, D), lambda i: (0, 0)),
            pl.BlockSpec((D, D), lambda i: (0, 0)),
        ],
        out_specs=pl.BlockSpec((_TC_R, D), lambda i: (i, 0)),
        out_shape=jax.ShapeDtypeStruct((N, D), jnp.float32),
    )(h, acc, deg, wl_t, b, wr_t)


def kernel(x, edge_index, W_l0, b_l0, W_r0, W_l1, b_l1, W_r1):
    src = edge_index[0].astype(jnp.int32)
    dst = edge_index[1].astype(jnp.int32)
    src = jnp.concatenate([src, jnp.zeros((EP - E,), jnp.int32)])
    dst = jnp.concatenate([dst, jnp.full((EP - E,), N, jnp.int32)])
    zeros = jnp.zeros((ZR, D), jnp.float32)
    ones = jnp.ones((C, D), jnp.float32)

    deg = _sc_deg(dst, zeros, ones)
    acc0 = _sc_agg(x, src, dst, zeros)
    h1 = _tc_layer(x, acc0, deg, W_l0.T, b_l0.reshape(1, D), W_r0.T)
    acc1 = _sc_agg(h1, src, dst, zeros)
    out = _tc_layer(h1, acc1, deg, W_l1.T, b_l1.reshape(1, D), W_r1.T)
    return out
